# baseline (speedup 1.0000x reference)
import jax
import jax.numpy as jnp
from jax.experimental import pallas as pl
from jax.experimental.pallas import tpu as pltpu

DIM = 512
CODEBOOK_SIZE = 1024
CODEBOOK_DIM = 64
NUM_CODEBOOKS = 2

ROW_TILE = 512


def _rpq_kernel(x_ref, p_ref, cbn_ref, out_ref):
    x = x_ref[...]
    p = p_ref[...]

    mu = jnp.mean(x, axis=-1, keepdims=True)
    xc = x - mu
    var = jnp.mean(xc * xc, axis=-1, keepdims=True)
    xn = xc / jnp.sqrt(var + 1e-5)

    proj = jnp.dot(xn, p, preferred_element_type=jnp.float32)

    idxs = []
    for h in range(NUM_CODEBOOKS):
        cbn = cbn_ref[h]
        ph = proj[:, h * CODEBOOK_DIM:(h + 1) * CODEBOOK_DIM]
        phn = ph / jnp.clip(
            jnp.sqrt(jnp.sum(ph * ph, axis=-1, keepdims=True)), 1e-12, None)
        scores = jnp.dot(phn, cbn.T, preferred_element_type=jnp.float32)
        idxs.append(jnp.argmax(scores, axis=-1).astype(jnp.int32))
    out_ref[...] = jnp.stack(idxs, axis=-1)


def kernel(x, rand_projs, embed):
    b, n, d = x.shape
    m = b * n
    xf = x.reshape(m, d)
    p = rand_projs.transpose(1, 0, 2).reshape(d, NUM_CODEBOOKS * CODEBOOK_DIM)
    cbn = embed / jnp.clip(
        jnp.sqrt(jnp.sum(embed * embed, axis=-1, keepdims=True)), 1e-12, None)

    grid = (m // ROW_TILE,)
    out_shape = jax.ShapeDtypeStruct((m, NUM_CODEBOOKS), jnp.int32)
    out = pl.pallas_call(
        _rpq_kernel,
        grid=grid,
        in_specs=[
            pl.BlockSpec((ROW_TILE, d), lambda i: (i, 0)),
            pl.BlockSpec((d, NUM_CODEBOOKS * CODEBOOK_DIM), lambda i: (0, 0)),
            pl.BlockSpec((NUM_CODEBOOKS, CODEBOOK_SIZE, CODEBOOK_DIM),
                         lambda i: (0, 0, 0)),
        ],
        out_specs=pl.BlockSpec((ROW_TILE, NUM_CODEBOOKS), lambda i: (i, 0)),
        out_shape=out_shape,
        compiler_params=pltpu.CompilerParams(
            dimension_semantics=("parallel",)),
    )(xf, p, cbn)
    return out.reshape(b, n, NUM_CODEBOOKS)


# speedup vs baseline: 1.1896x; 1.1896x over previous
import jax
import jax.numpy as jnp
from jax.experimental import pallas as pl
from jax.experimental.pallas import tpu as pltpu

DIM = 512
CODEBOOK_SIZE = 1024
CODEBOOK_DIM = 64
NUM_CODEBOOKS = 2

ROW_TILE = 1024


def _rpq_kernel(x_ref, p_ref, cbt_ref, out_ref):
    x = x_ref[...]
    p = p_ref[...]

    mu = jnp.mean(x, axis=-1, keepdims=True)
    xc = x - mu
    var = jnp.mean(xc * xc, axis=-1, keepdims=True)
    xn = xc / jnp.sqrt(var + 1e-5)

    proj = jnp.dot(xn, p, preferred_element_type=jnp.float32)

    phns = []
    for h in range(NUM_CODEBOOKS):
        ph = proj[:, h * CODEBOOK_DIM:(h + 1) * CODEBOOK_DIM]
        phns.append(ph / jnp.clip(
            jnp.sqrt(jnp.sum(ph * ph, axis=-1, keepdims=True)), 1e-12, None))
    phn = jnp.concatenate(phns, axis=-1)

    scores = jnp.dot(phn, cbt_ref[...], preferred_element_type=jnp.float32)
    idxs = [jnp.argmax(scores[:, h * CODEBOOK_SIZE:(h + 1) * CODEBOOK_SIZE],
                       axis=-1).astype(jnp.int32)
            for h in range(NUM_CODEBOOKS)]
    out_ref[...] = jnp.stack(idxs, axis=-1)


def kernel(x, rand_projs, embed):
    b, n, d = x.shape
    m = b * n
    xf = x.reshape(m, d)
    p = rand_projs.transpose(1, 0, 2).reshape(d, NUM_CODEBOOKS * CODEBOOK_DIM)
    cbn = embed / jnp.clip(
        jnp.sqrt(jnp.sum(embed * embed, axis=-1, keepdims=True)), 1e-12, None)
    cbt = jnp.zeros((NUM_CODEBOOKS * CODEBOOK_DIM,
                     NUM_CODEBOOKS * CODEBOOK_SIZE), jnp.float32)
    for h in range(NUM_CODEBOOKS):
        cbt = cbt.at[h * CODEBOOK_DIM:(h + 1) * CODEBOOK_DIM,
                     h * CODEBOOK_SIZE:(h + 1) * CODEBOOK_SIZE].set(cbn[h].T)

    grid = (m // ROW_TILE,)
    out_shape = jax.ShapeDtypeStruct((m, NUM_CODEBOOKS), jnp.int32)
    out = pl.pallas_call(
        _rpq_kernel,
        grid=grid,
        in_specs=[
            pl.BlockSpec((ROW_TILE, d), lambda i: (i, 0)),
            pl.BlockSpec((d, NUM_CODEBOOKS * CODEBOOK_DIM), lambda i: (0, 0)),
            pl.BlockSpec((NUM_CODEBOOKS * CODEBOOK_DIM,
                          NUM_CODEBOOKS * CODEBOOK_SIZE),
                         lambda i: (0, 0)),
        ],
        out_specs=pl.BlockSpec((ROW_TILE, NUM_CODEBOOKS), lambda i: (i, 0)),
        out_shape=out_shape,
        compiler_params=pltpu.CompilerParams(
            dimension_semantics=("parallel",)),
    )(xf, p, cbt)
    return out.reshape(b, n, NUM_CODEBOOKS)


# speedup vs baseline: 1.2466x; 1.0479x over previous
import jax
import jax.numpy as jnp
from jax.experimental import pallas as pl
from jax.experimental.pallas import tpu as pltpu

DIM = 512
CODEBOOK_SIZE = 1024
CODEBOOK_DIM = 64
NUM_CODEBOOKS = 2

ROW_TILE = 2048


def _rpq_kernel(x_ref, p_ref, cbt_ref, out_ref):
    x = x_ref[...]
    p = p_ref[...]

    mu = jnp.mean(x, axis=-1, keepdims=True)
    xc = x - mu
    var = jnp.mean(xc * xc, axis=-1, keepdims=True)
    xn = xc / jnp.sqrt(var + 1e-5)

    proj = jnp.dot(xn, p, preferred_element_type=jnp.float32)

    phns = []
    for h in range(NUM_CODEBOOKS):
        ph = proj[:, h * CODEBOOK_DIM:(h + 1) * CODEBOOK_DIM]
        phns.append(ph / jnp.clip(
            jnp.sqrt(jnp.sum(ph * ph, axis=-1, keepdims=True)), 1e-12, None))
    phn = jnp.concatenate(phns, axis=-1)

    scores = jnp.dot(phn, cbt_ref[...], preferred_element_type=jnp.float32)
    idxs = [jnp.argmax(scores[:, h * CODEBOOK_SIZE:(h + 1) * CODEBOOK_SIZE],
                       axis=-1).astype(jnp.int32)
            for h in range(NUM_CODEBOOKS)]
    out_ref[...] = jnp.stack(idxs, axis=-1)


def kernel(x, rand_projs, embed):
    b, n, d = x.shape
    m = b * n
    xf = x.reshape(m, d)
    p = rand_projs.transpose(1, 0, 2).reshape(d, NUM_CODEBOOKS * CODEBOOK_DIM)
    cbn = embed / jnp.clip(
        jnp.sqrt(jnp.sum(embed * embed, axis=-1, keepdims=True)), 1e-12, None)
    cbt = jnp.zeros((NUM_CODEBOOKS * CODEBOOK_DIM,
                     NUM_CODEBOOKS * CODEBOOK_SIZE), jnp.float32)
    for h in range(NUM_CODEBOOKS):
        cbt = cbt.at[h * CODEBOOK_DIM:(h + 1) * CODEBOOK_DIM,
                     h * CODEBOOK_SIZE:(h + 1) * CODEBOOK_SIZE].set(cbn[h].T)

    grid = (m // ROW_TILE,)
    out_shape = jax.ShapeDtypeStruct((m, NUM_CODEBOOKS), jnp.int32)
    out = pl.pallas_call(
        _rpq_kernel,
        grid=grid,
        in_specs=[
            pl.BlockSpec((ROW_TILE, d), lambda i: (i, 0)),
            pl.BlockSpec((d, NUM_CODEBOOKS * CODEBOOK_DIM), lambda i: (0, 0)),
            pl.BlockSpec((NUM_CODEBOOKS * CODEBOOK_DIM,
                          NUM_CODEBOOKS * CODEBOOK_SIZE),
                         lambda i: (0, 0)),
        ],
        out_specs=pl.BlockSpec((ROW_TILE, NUM_CODEBOOKS), lambda i: (i, 0)),
        out_shape=out_shape,
        compiler_params=pltpu.CompilerParams(
            dimension_semantics=("parallel",)),
    )(xf, p, cbt)
    return out.reshape(b, n, NUM_CODEBOOKS)


# speedup vs baseline: 1.2526x; 1.0048x over previous
import jax
import jax.numpy as jnp
from jax.experimental import pallas as pl
from jax.experimental.pallas import tpu as pltpu

DIM = 512
CODEBOOK_SIZE = 1024
CODEBOOK_DIM = 64
NUM_CODEBOOKS = 2

ROW_TILE = 4096


def _rpq_kernel(x_ref, p_ref, cbt_ref, out_ref):
    x = x_ref[...]
    p = p_ref[...]

    mu = jnp.mean(x, axis=-1, keepdims=True)
    xc = x - mu
    var = jnp.mean(xc * xc, axis=-1, keepdims=True)
    xn = xc / jnp.sqrt(var + 1e-5)

    proj = jnp.dot(xn, p, preferred_element_type=jnp.float32)

    phns = []
    for h in range(NUM_CODEBOOKS):
        ph = proj[:, h * CODEBOOK_DIM:(h + 1) * CODEBOOK_DIM]
        phns.append(ph / jnp.clip(
            jnp.sqrt(jnp.sum(ph * ph, axis=-1, keepdims=True)), 1e-12, None))
    phn = jnp.concatenate(phns, axis=-1)

    scores = jnp.dot(phn, cbt_ref[...], preferred_element_type=jnp.float32)
    idxs = [jnp.argmax(scores[:, h * CODEBOOK_SIZE:(h + 1) * CODEBOOK_SIZE],
                       axis=-1).astype(jnp.int32)
            for h in range(NUM_CODEBOOKS)]
    out_ref[...] = jnp.stack(idxs, axis=-1)


def kernel(x, rand_projs, embed):
    b, n, d = x.shape
    m = b * n
    xf = x.reshape(m, d)
    p = rand_projs.transpose(1, 0, 2).reshape(d, NUM_CODEBOOKS * CODEBOOK_DIM)
    cbn = embed / jnp.clip(
        jnp.sqrt(jnp.sum(embed * embed, axis=-1, keepdims=True)), 1e-12, None)
    cbt = jnp.zeros((NUM_CODEBOOKS * CODEBOOK_DIM,
                     NUM_CODEBOOKS * CODEBOOK_SIZE), jnp.float32)
    for h in range(NUM_CODEBOOKS):
        cbt = cbt.at[h * CODEBOOK_DIM:(h + 1) * CODEBOOK_DIM,
                     h * CODEBOOK_SIZE:(h + 1) * CODEBOOK_SIZE].set(cbn[h].T)

    grid = (m // ROW_TILE,)
    out_shape = jax.ShapeDtypeStruct((m, NUM_CODEBOOKS), jnp.int32)
    out = pl.pallas_call(
        _rpq_kernel,
        grid=grid,
        in_specs=[
            pl.BlockSpec((ROW_TILE, d), lambda i: (i, 0)),
            pl.BlockSpec((d, NUM_CODEBOOKS * CODEBOOK_DIM), lambda i: (0, 0)),
            pl.BlockSpec((NUM_CODEBOOKS * CODEBOOK_DIM,
                          NUM_CODEBOOKS * CODEBOOK_SIZE),
                         lambda i: (0, 0)),
        ],
        out_specs=pl.BlockSpec((ROW_TILE, NUM_CODEBOOKS), lambda i: (i, 0)),
        out_shape=out_shape,
        compiler_params=pltpu.CompilerParams(
            dimension_semantics=("parallel",)),
    )(xf, p, cbt)
    return out.reshape(b, n, NUM_CODEBOOKS)
